# Initial kernel scaffold; baseline (speedup 1.0000x reference)
import functools

import jax
import jax.numpy as jnp
from jax import lax
from jax.experimental import pallas as pl
from jax.experimental.pallas import tpu as pltpu
from jax.experimental.pallas import tpu_sc as plsc

N = 10000
E = 320000
D = 128

NC = 2
NS = 16
NW = NC * NS
EPW = E // NW
CH = 125
NCHUNK = EPW // CH
RPW = N // NS
DEGW = 16

_mesh = plsc.VectorSubcoreMesh(core_axis_name="c", subcore_axis_name="s")


@functools.partial(
    pl.kernel,
    out_type=jax.ShapeDtypeStruct((NC, N, DEGW), jnp.float32),
    mesh=_mesh,
    scratch_types=[
        pltpu.VMEM((NCHUNK, CH), jnp.int32),
        pltpu.VMEM((CH, DEGW), jnp.float32),
        pltpu.VMEM((RPW, DEGW), jnp.float32),
        pltpu.VMEM_SHARED((N, DEGW), jnp.float32),
        pltpu.SemaphoreType.DMA,
    ],
)
def _deg_kernel(dst_hbm, out_hbm, idx_v, ones_v, zb_v, deg_sh, sem):
    c = lax.axis_index("c")
    s = lax.axis_index("s")
    wid = s * NC + c
    pltpu.sync_copy(dst_hbm.at[wid], idx_v)

    def fill_ones(i, carry):
        ones_v[i, :] = jnp.ones((DEGW,), jnp.float32)
        return carry

    lax.fori_loop(0, CH, fill_ones, 0)

    def fill_zero(i, carry):
        zb_v[i, :] = jnp.zeros((DEGW,), jnp.float32)
        return carry

    lax.fori_loop(0, RPW, fill_zero, 0)

    pltpu.sync_copy(zb_v, deg_sh.at[pl.ds(s * RPW, RPW), :])
    plsc.subcore_barrier()

    def body(j, carry):
        pltpu.sync_copy(ones_v, deg_sh.at[idx_v.at[j]], add=True)
        return carry

    lax.fori_loop(0, NCHUNK, body, 0)
    plsc.subcore_barrier()

    pltpu.sync_copy(deg_sh.at[pl.ds(s * RPW, RPW), :], zb_v)
    pltpu.sync_copy(zb_v, out_hbm.at[c, pl.ds(s * RPW, RPW), :])


@functools.partial(
    pl.kernel,
    out_type=jax.ShapeDtypeStruct((NC, N, D), jnp.float32),
    mesh=_mesh,
    scratch_types=[
        pltpu.VMEM((NCHUNK, CH), jnp.int32),
        pltpu.VMEM((NCHUNK, CH), jnp.int32),
        pltpu.VMEM((CH, D), jnp.float32),
        pltpu.VMEM((CH, D), jnp.float32),
        pltpu.VMEM_SHARED((N, D), jnp.float32),
        pltpu.SemaphoreType.DMA,
    ],
)
def _scatter_kernel(y_hbm, src_hbm, dst_hbm, out_hbm,
                    si_v, di_v, r0, r1, acc_sh, sem):
    c = lax.axis_index("c")
    s = lax.axis_index("s")
    wid = s * NC + c
    pltpu.sync_copy(src_hbm.at[wid], si_v)
    pltpu.sync_copy(dst_hbm.at[wid], di_v)

    def zrow(i, carry):
        for k8 in range(D // 16):
            r0[i, pl.ds(k8 * 16, 16)] = jnp.zeros((16,), jnp.float32)
        return carry

    lax.fori_loop(0, CH, zrow, 0)

    def zcopy(k, carry):
        pltpu.sync_copy(r0, acc_sh.at[pl.ds(s * RPW + k * CH, CH), :])
        return carry

    lax.fori_loop(0, RPW // CH, zcopy, 0)
    plsc.subcore_barrier()

    def body(j, carry):
        pltpu.async_copy(y_hbm.at[si_v.at[j]], r0, sem).wait()
        pltpu.sync_copy(r0, acc_sh.at[di_v.at[j]], add=True)
        return carry

    lax.fori_loop(0, NCHUNK, body, 0)
    plsc.subcore_barrier()

    def ocopy(k, carry):
        pltpu.sync_copy(acc_sh.at[pl.ds(s * RPW + k * CH, CH), :], r1)
        pltpu.sync_copy(r1, out_hbm.at[c, pl.ds(s * RPW + k * CH, CH), :])
        return carry

    lax.fori_loop(0, RPW // CH, ocopy, 0)


_R = 2000


def _tc_a_body(deg_ref, x_ref, w_ref, dinv_ref, y_ref):
    deg = deg_ref[0, :, 0:1] + deg_ref[1, :, 0:1] + 1.0
    di = lax.rsqrt(deg)
    dinv_ref[...] = di
    y_ref[...] = di * jnp.dot(x_ref[...], w_ref[...],
                              preferred_element_type=jnp.float32)


def _tc_b_body(s_ref, y_ref, dinv_ref, b_ref, w_ref, y2_ref):
    di = dinv_ref[...]
    sm = s_ref[0] + s_ref[1] + y_ref[...]
    h = jnp.maximum(di * sm + b_ref[...], 0.0)
    y2_ref[...] = di * jnp.dot(h, w_ref[...],
                               preferred_element_type=jnp.float32)


def _tc_c_body(s_ref, y_ref, dinv_ref, b_ref, o_ref):
    sm = s_ref[0] + s_ref[1] + y_ref[...]
    o_ref[...] = dinv_ref[...] * sm + b_ref[...]


def _tc_a(deg2, x, W1):
    grid = (N // _R,)
    return pl.pallas_call(
        _tc_a_body,
        grid=grid,
        in_specs=[
            pl.BlockSpec((NC, _R, DEGW), lambda i: (0, i, 0)),
            pl.BlockSpec((_R, D), lambda i: (i, 0)),
            pl.BlockSpec((D, D), lambda i: (0, 0)),
        ],
        out_specs=[
            pl.BlockSpec((_R, 1), lambda i: (i, 0)),
            pl.BlockSpec((_R, D), lambda i: (i, 0)),
        ],
        out_shape=[
            jax.ShapeDtypeStruct((N, 1), jnp.float32),
            jax.ShapeDtypeStruct((N, D), jnp.float32),
        ],
    )(deg2, x, W1)


def _tc_b(S1, y1, dinv, b1, W2):
    grid = (N // _R,)
    return pl.pallas_call(
        _tc_b_body,
        grid=grid,
        in_specs=[
            pl.BlockSpec((NC, _R, D), lambda i: (0, i, 0)),
            pl.BlockSpec((_R, D), lambda i: (i, 0)),
            pl.BlockSpec((_R, 1), lambda i: (i, 0)),
            pl.BlockSpec((1, D), lambda i: (0, 0)),
            pl.BlockSpec((D, D), lambda i: (0, 0)),
        ],
        out_specs=pl.BlockSpec((_R, D), lambda i: (i, 0)),
        out_shape=jax.ShapeDtypeStruct((N, D), jnp.float32),
    )(S1, y1, dinv, b1, W2)


def _tc_c(S2, y2, dinv, b2):
    grid = (N // _R,)
    return pl.pallas_call(
        _tc_c_body,
        grid=grid,
        in_specs=[
            pl.BlockSpec((NC, _R, D), lambda i: (0, i, 0)),
            pl.BlockSpec((_R, D), lambda i: (i, 0)),
            pl.BlockSpec((_R, 1), lambda i: (i, 0)),
            pl.BlockSpec((1, D), lambda i: (0, 0)),
        ],
        out_specs=pl.BlockSpec((_R, D), lambda i: (i, 0)),
        out_shape=jax.ShapeDtypeStruct((N, D), jnp.float32),
    )(S2, y2, dinv, b2)


def kernel(x, edge_index, W1, b1, W2, b2):
    src = edge_index[0].astype(jnp.int32).reshape(NW, NCHUNK, CH)
    dst = edge_index[1].astype(jnp.int32).reshape(NW, NCHUNK, CH)
    deg2 = _deg_kernel(dst)
    dinv, y1 = _tc_a(deg2, x, W1)
    S1 = _scatter_kernel(y1, src, dst)
    y2 = _tc_b(S1, y1, dinv, b1.reshape(1, D), W2)
    S2 = _scatter_kernel(y2, src, dst)
    return _tc_c(S2, y2, dinv, b2.reshape(1, D))


# speedup vs baseline: 17.2046x; 17.2046x over previous
import functools

import jax
import jax.numpy as jnp
from jax import lax
from jax.experimental import pallas as pl
from jax.experimental.pallas import tpu as pltpu
from jax.experimental.pallas import tpu_sc as plsc

N = 10000
NP = 10240
E = 320000
D = 128

NC = 2
NS = 16
NW = NC * NS
EPW = E // NW
CH = 80
NCHUNK = EPW // CH
RPW = NP // NS
OCH = 32
DEGW = 16


def _deg_body(dst_hbm, out_hbm, idx_v, ones_v, zb_v, deg_sh, sem):
    c = lax.axis_index("c")
    s = lax.axis_index("s")
    wid = s * NC + c
    pltpu.sync_copy(dst_hbm.at[wid], idx_v)

    for i in range(CH // 16):
        ones_v[pl.ds(i * 16, 16)] = jnp.ones((16,), jnp.float32)

    def fill_zero(i, carry):
        zb_v[pl.ds(i * 16, 16)] = jnp.zeros((16,), jnp.float32)
        return carry

    lax.fori_loop(0, RPW // 16, fill_zero, 0)

    pltpu.sync_copy(zb_v, deg_sh.at[pl.ds(s * RPW, RPW)])
    plsc.subcore_barrier()

    def body(j, carry):
        pltpu.sync_copy(ones_v, deg_sh.at[idx_v.at[j]], add=True)
        return carry

    lax.fori_loop(0, NCHUNK, body, 0)
    plsc.subcore_barrier()

    pltpu.sync_copy(deg_sh.at[pl.ds(s * RPW, RPW)], zb_v)
    pltpu.sync_copy(zb_v, out_hbm.at[c, pl.ds(s * RPW, RPW)])


def _scatter_body(y_hbm, src_hbm, dst_hbm, out_hbm,
                  si_v, di_v, r0, ob, acc_sh, sem):
    c = lax.axis_index("c")
    s = lax.axis_index("s")
    wid = s * NC + c
    pltpu.sync_copy(src_hbm.at[wid], si_v)
    pltpu.sync_copy(dst_hbm.at[wid], di_v)

    def zrow(i, carry):
        for k8 in range(D // 16):
            ob[i, pl.ds(k8 * 16, 16)] = jnp.zeros((16,), jnp.float32)
        return carry

    lax.fori_loop(0, OCH, zrow, 0)

    def zcopy(k, carry):
        pltpu.sync_copy(ob, acc_sh.at[pl.ds(s * RPW + k * OCH, OCH), :])
        return carry

    lax.fori_loop(0, RPW // OCH, zcopy, 0)
    plsc.subcore_barrier()

    def body(j, carry):
        pltpu.async_copy(y_hbm.at[si_v.at[j]], r0, sem).wait()
        pltpu.sync_copy(r0, acc_sh.at[di_v.at[j]], add=True)
        return carry

    lax.fori_loop(0, NCHUNK, body, 0)
    plsc.subcore_barrier()

    def ocopy(k, carry):
        pltpu.sync_copy(acc_sh.at[pl.ds(s * RPW + k * OCH, OCH), :], ob)
        pltpu.sync_copy(ob, out_hbm.at[c, pl.ds(s * RPW + k * OCH, OCH), :])
        return carry

    lax.fori_loop(0, RPW // OCH, ocopy, 0)


@functools.cache
def _sc_kernels():
    mesh = plsc.VectorSubcoreMesh(
        core_axis_name="c", subcore_axis_name="s", num_cores=NC)
    deg_k = pl.kernel(
        _deg_body,
        out_type=jax.ShapeDtypeStruct((NC, NP), jnp.float32),
        mesh=mesh,
        scratch_types=[
            pltpu.VMEM((NCHUNK, CH), jnp.int32),
            pltpu.VMEM((CH,), jnp.float32),
            pltpu.VMEM((RPW,), jnp.float32),
            pltpu.VMEM_SHARED((NP,), jnp.float32),
            pltpu.SemaphoreType.DMA,
        ],
    )
    scat_k = pl.kernel(
        _scatter_body,
        out_type=jax.ShapeDtypeStruct((NC, NP, D), jnp.float32),
        mesh=mesh,
        scratch_types=[
            pltpu.VMEM((NCHUNK, CH), jnp.int32),
            pltpu.VMEM((NCHUNK, CH), jnp.int32),
            pltpu.VMEM((CH, D), jnp.float32),
            pltpu.VMEM((OCH, D), jnp.float32),
            pltpu.VMEM_SHARED((NP, D), jnp.float32),
            pltpu.SemaphoreType.DMA,
        ],
    )
    return deg_k, scat_k


_R = 2048


def _tc_a_body(dinv_ref, x_ref, w_ref, y_ref):
    y_ref[...] = dinv_ref[...] * jnp.dot(x_ref[...], w_ref[...],
                                         preferred_element_type=jnp.float32)


def _tc_b_body(s_ref, y_ref, dinv_ref, b_ref, w_ref, y2_ref):
    di = dinv_ref[...]
    sm = s_ref[0] + s_ref[1] + y_ref[...]
    h = jnp.maximum(di * sm + b_ref[...], 0.0)
    y2_ref[...] = di * jnp.dot(h, w_ref[...],
                               preferred_element_type=jnp.float32)


def _tc_c_body(s_ref, y_ref, dinv_ref, b_ref, o_ref):
    sm = s_ref[0] + s_ref[1] + y_ref[...]
    o_ref[...] = dinv_ref[...] * sm + b_ref[...]


def _tc_a(dinv, x, W1):
    return pl.pallas_call(
        _tc_a_body,
        grid=(NP // _R,),
        in_specs=[
            pl.BlockSpec((_R, 1), lambda i: (i, 0)),
            pl.BlockSpec((_R, D), lambda i: (i, 0)),
            pl.BlockSpec((D, D), lambda i: (0, 0)),
        ],
        out_specs=pl.BlockSpec((_R, D), lambda i: (i, 0)),
        out_shape=jax.ShapeDtypeStruct((NP, D), jnp.float32),
    )(dinv, x, W1)


def _tc_b(S1, y1, dinv, b1, W2):
    return pl.pallas_call(
        _tc_b_body,
        grid=(NP // _R,),
        in_specs=[
            pl.BlockSpec((NC, _R, D), lambda i: (0, i, 0)),
            pl.BlockSpec((_R, D), lambda i: (i, 0)),
            pl.BlockSpec((_R, 1), lambda i: (i, 0)),
            pl.BlockSpec((1, D), lambda i: (0, 0)),
            pl.BlockSpec((D, D), lambda i: (0, 0)),
        ],
        out_specs=pl.BlockSpec((_R, D), lambda i: (i, 0)),
        out_shape=jax.ShapeDtypeStruct((NP, D), jnp.float32),
    )(S1, y1, dinv, b1, W2)


def _tc_c(S2, y2, dinv, b2):
    return pl.pallas_call(
        _tc_c_body,
        grid=(NP // _R,),
        in_specs=[
            pl.BlockSpec((NC, _R, D), lambda i: (0, i, 0)),
            pl.BlockSpec((_R, D), lambda i: (i, 0)),
            pl.BlockSpec((_R, 1), lambda i: (i, 0)),
            pl.BlockSpec((1, D), lambda i: (0, 0)),
        ],
        out_specs=pl.BlockSpec((_R, D), lambda i: (i, 0)),
        out_shape=jax.ShapeDtypeStruct((NP, D), jnp.float32),
    )(S2, y2, dinv, b2)


def kernel(x, edge_index, W1, b1, W2, b2):
    deg_k, scat_k = _sc_kernels()
    src = edge_index[0].astype(jnp.int32).reshape(NW, NCHUNK, CH)
    dst = edge_index[1].astype(jnp.int32).reshape(NW, NCHUNK, CH)
    xp = jnp.pad(x, ((0, NP - N), (0, 0)))
    deg2 = deg_k(dst)
    dinv = lax.rsqrt(deg2[0] + deg2[1] + 1.0)[:, None]
    y1 = _tc_a(dinv, xp, W1)
    S1 = scat_k(y1, src, dst)
    y2 = _tc_b(S1, y1, dinv, b1.reshape(1, D), W2)
    S2 = scat_k(y2, src, dst)
    return _tc_c(S2, y2, dinv, b2.reshape(1, D))[:N]


# speedup vs baseline: 17.2706x; 1.0038x over previous
import functools

import jax
import jax.numpy as jnp
from jax import lax
from jax.experimental import pallas as pl
from jax.experimental.pallas import tpu as pltpu
from jax.experimental.pallas import tpu_sc as plsc

N = 10000
NP = 10240
E = 320000
D = 128

NC = 2
NS = 16
NW = NC * NS
EPW = E // NW
CH = 80
NCHUNK = EPW // CH
RPW = NP // NS
OCH = 32
DEGW = 16


def _deg_body(dst_hbm, out_hbm, idx_v, ones_v, zb_v, deg_sh, sem):
    c = lax.axis_index("c")
    s = lax.axis_index("s")
    wid = s * NC + c
    pltpu.sync_copy(dst_hbm.at[wid], idx_v)

    for i in range(CH // 16):
        ones_v[pl.ds(i * 16, 16)] = jnp.ones((16,), jnp.float32)

    def fill_zero(i, carry):
        zb_v[pl.ds(i * 16, 16)] = jnp.zeros((16,), jnp.float32)
        return carry

    lax.fori_loop(0, RPW // 16, fill_zero, 0)

    pltpu.sync_copy(zb_v, deg_sh.at[pl.ds(s * RPW, RPW)])
    plsc.subcore_barrier()

    def body(j, carry):
        pltpu.sync_copy(ones_v, deg_sh.at[idx_v.at[j]], add=True)
        return carry

    lax.fori_loop(0, NCHUNK, body, 0)
    plsc.subcore_barrier()

    pltpu.sync_copy(deg_sh.at[pl.ds(s * RPW, RPW)], zb_v)
    pltpu.sync_copy(zb_v, out_hbm.at[c, pl.ds(s * RPW, RPW)])


def _scatter_body(y_hbm, pk_hbm, out_hbm,
                  pk_v, si2, di2, r0, r1, acc_sh, sem0, sem1):
    c = lax.axis_index("c")
    s = lax.axis_index("s")
    wid = s * NC + c
    pltpu.sync_copy(pk_hbm.at[wid], pk_v)

    bufs = ((r0, si2.at[0], di2.at[0], sem0), (r1, si2.at[1], di2.at[1], sem1))

    def zrow(i, carry):
        for k8 in range(D // 16):
            r0[i, pl.ds(k8 * 16, 16)] = jnp.zeros((16,), jnp.float32)
        return carry

    lax.fori_loop(0, CH, zrow, 0)

    def zcopy(k, carry):
        pltpu.sync_copy(r0, acc_sh.at[pl.ds(s * RPW + k * CH, CH), :])
        return carry

    lax.fori_loop(0, RPW // CH, zcopy, 0)
    plsc.subcore_barrier()

    def unpack(j, b):
        for i in range(CH // 16):
            v = pk_v[j, pl.ds(i * 16, 16)]
            si2[b, pl.ds(i * 16, 16)] = v & 0xFFFF
            di2[b, pl.ds(i * 16, 16)] = v >> 16

    def body(j, carry):
        unpack(j, 0)
        pltpu.async_copy(y_hbm.at[si2.at[0]], r0, sem0).wait()
        pltpu.sync_copy(r0, acc_sh.at[di2.at[0]], add=True)
        return carry

    lax.fori_loop(0, NCHUNK, body, 0)
    plsc.subcore_barrier()

    def ocopy(k, carry):
        pltpu.sync_copy(acc_sh.at[pl.ds(s * RPW + k * CH, CH), :], r1)
        pltpu.sync_copy(r1, out_hbm.at[c, pl.ds(s * RPW + k * CH, CH), :])
        return carry

    lax.fori_loop(0, RPW // CH, ocopy, 0)


@functools.cache
def _sc_kernels():
    mesh = plsc.VectorSubcoreMesh(
        core_axis_name="c", subcore_axis_name="s", num_cores=NC)
    deg_k = pl.kernel(
        _deg_body,
        out_type=jax.ShapeDtypeStruct((NC, NP), jnp.float32),
        mesh=mesh,
        scratch_types=[
            pltpu.VMEM((NCHUNK, CH), jnp.int32),
            pltpu.VMEM((CH,), jnp.float32),
            pltpu.VMEM((RPW,), jnp.float32),
            pltpu.VMEM_SHARED((NP,), jnp.float32),
            pltpu.SemaphoreType.DMA,
        ],
    )
    scat_k = pl.kernel(
        _scatter_body,
        out_type=jax.ShapeDtypeStruct((NC, NP, D), jnp.float32),
        mesh=mesh,
        scratch_types=[
            pltpu.VMEM((NCHUNK, CH), jnp.int32),
            pltpu.VMEM((2, CH), jnp.int32),
            pltpu.VMEM((2, CH), jnp.int32),
            pltpu.VMEM((CH, D), jnp.float32),
            pltpu.VMEM((CH, D), jnp.float32),
            pltpu.VMEM_SHARED((NP, D), jnp.float32),
            pltpu.SemaphoreType.DMA,
            pltpu.SemaphoreType.DMA,
        ],
    )
    return deg_k, scat_k


_R = 2048


def _tc_a_body(dinv_ref, x_ref, w_ref, y_ref):
    y_ref[...] = dinv_ref[...] * jnp.dot(x_ref[...], w_ref[...],
                                         preferred_element_type=jnp.float32)


def _tc_b_body(s_ref, y_ref, dinv_ref, b_ref, w_ref, y2_ref):
    di = dinv_ref[...]
    sm = s_ref[0] + s_ref[1] + y_ref[...]
    h = jnp.maximum(di * sm + b_ref[...], 0.0)
    y2_ref[...] = di * jnp.dot(h, w_ref[...],
                               preferred_element_type=jnp.float32)


def _tc_c_body(s_ref, y_ref, dinv_ref, b_ref, o_ref):
    sm = s_ref[0] + s_ref[1] + y_ref[...]
    o_ref[...] = dinv_ref[...] * sm + b_ref[...]


def _tc_a(dinv, x, W1):
    return pl.pallas_call(
        _tc_a_body,
        grid=(NP // _R,),
        in_specs=[
            pl.BlockSpec((_R, 1), lambda i: (i, 0)),
            pl.BlockSpec((_R, D), lambda i: (i, 0)),
            pl.BlockSpec((D, D), lambda i: (0, 0)),
        ],
        out_specs=pl.BlockSpec((_R, D), lambda i: (i, 0)),
        out_shape=jax.ShapeDtypeStruct((NP, D), jnp.float32),
    )(dinv, x, W1)


def _tc_b(S1, y1, dinv, b1, W2):
    return pl.pallas_call(
        _tc_b_body,
        grid=(NP // _R,),
        in_specs=[
            pl.BlockSpec((NC, _R, D), lambda i: (0, i, 0)),
            pl.BlockSpec((_R, D), lambda i: (i, 0)),
            pl.BlockSpec((_R, 1), lambda i: (i, 0)),
            pl.BlockSpec((1, D), lambda i: (0, 0)),
            pl.BlockSpec((D, D), lambda i: (0, 0)),
        ],
        out_specs=pl.BlockSpec((_R, D), lambda i: (i, 0)),
        out_shape=jax.ShapeDtypeStruct((NP, D), jnp.float32),
    )(S1, y1, dinv, b1, W2)


def _tc_c(S2, y2, dinv, b2):
    return pl.pallas_call(
        _tc_c_body,
        grid=(NP // _R,),
        in_specs=[
            pl.BlockSpec((NC, _R, D), lambda i: (0, i, 0)),
            pl.BlockSpec((_R, D), lambda i: (i, 0)),
            pl.BlockSpec((_R, 1), lambda i: (i, 0)),
            pl.BlockSpec((1, D), lambda i: (0, 0)),
        ],
        out_specs=pl.BlockSpec((_R, D), lambda i: (i, 0)),
        out_shape=jax.ShapeDtypeStruct((NP, D), jnp.float32),
    )(S2, y2, dinv, b2)


def kernel(x, edge_index, W1, b1, W2, b2):
    deg_k, scat_k = _sc_kernels()
    src = edge_index[0].astype(jnp.int32).reshape(NW, NCHUNK, CH)
    dst = edge_index[1].astype(jnp.int32).reshape(NW, NCHUNK, CH)
    xp = jnp.pad(x, ((0, NP - N), (0, 0)))
    packed = src | (dst << 16)
    deg2 = deg_k(dst)
    dinv = lax.rsqrt(deg2[0] + deg2[1] + 1.0)[:, None]
    y1 = _tc_a(dinv, xp, W1)
    S1 = scat_k(y1, packed)
    y2 = _tc_b(S1, y1, dinv, b1.reshape(1, D), W2)
    S2 = scat_k(y2, packed)
    return _tc_c(S2, y2, dinv, b2.reshape(1, D))[:N]
